# Initial kernel scaffold; baseline (speedup 1.0000x reference)
import jax
import jax.numpy as jnp
from jax.experimental import pallas as pl


def kernel(x, edge_index, batch, gn0_w, gn0_b, gn0_a, W1r, b1r, W1t, gn1_w, gn1_b, gn1_a, W2r, b2r, W2t, gn2_w, gn2_b, gn2_a, W3r, b3r, W3t, dW, db, oW, ob):
    raise NotImplementedError("write your pallas kernel here")


# speedup vs baseline: 6.7844x; 6.7844x over previous
import functools

import jax
import jax.numpy as jnp
from jax import lax
from jax.experimental import pallas as pl
from jax.experimental.pallas import tpu as pltpu
from jax.experimental.pallas import tpu_sc as plsc

N = 10000
E = 320000
F = 128
G = 8
C = 10

NC = 2
NS = 16
FH = F // NC
EPT = E // NS
CH = 100
NCHUNK = EPT // CH
NP = 10240
RPT = NP // NS

def _sc_agg_body(h_hbm, src_hbm, dst_hbm, zero_hbm, out_hbm,
                 src_v, dst_v, buf0, buf1, acc, gsem):
    cid = lax.axis_index("c")
    sid = lax.axis_index("s")

    pltpu.sync_copy(zero_hbm, acc.at[pl.ds(sid * RPT, RPT)])
    pltpu.sync_copy(src_hbm.at[sid], src_v)
    pltpu.sync_copy(dst_hbm.at[sid], dst_v)
    plsc.subcore_barrier()

    hsrc = h_hbm.at[cid]

    pltpu.async_copy(hsrc.at[src_v.at[0]], buf0, gsem)

    def body(it, carry):
        i0 = it * 2
        i1 = i0 + 1
        pltpu.make_async_copy(hsrc.at[src_v.at[i0]], buf0, gsem).wait()
        pltpu.async_copy(hsrc.at[src_v.at[i1]], buf1, gsem)
        pltpu.sync_copy(buf0, acc.at[dst_v.at[i0]], add=True)
        pltpu.make_async_copy(hsrc.at[src_v.at[i1]], buf1, gsem).wait()

        @pl.when(i1 + 1 < NCHUNK)
        def _():
            pltpu.async_copy(hsrc.at[src_v.at[i1 + 1]], buf0, gsem)

        pltpu.sync_copy(buf1, acc.at[dst_v.at[i1]], add=True)
        return carry

    lax.fori_loop(0, NCHUNK // 2, body, 0)

    plsc.subcore_barrier()
    pltpu.sync_copy(acc.at[pl.ds(sid * RPT, RPT)],
                    out_hbm.at[cid, pl.ds(sid * RPT, RPT)])


@functools.lru_cache(maxsize=1)
def _build_sc_agg():
    mesh = plsc.VectorSubcoreMesh(
        core_axis_name="c", subcore_axis_name="s",
        num_cores=NC, num_subcores=NS)
    return pl.kernel(
        _sc_agg_body,
        out_type=jax.ShapeDtypeStruct((NC, NP, FH), jnp.float32),
        mesh=mesh,
        scratch_types=[
            pltpu.VMEM((NCHUNK, CH), jnp.int32),
            pltpu.VMEM((NCHUNK, CH), jnp.int32),
            pltpu.VMEM((CH, FH), jnp.float32),
            pltpu.VMEM((CH, FH), jnp.float32),
            pltpu.VMEM_SHARED((NP, FH), jnp.float32),
            pltpu.SemaphoreType.DMA,
        ],
        compiler_params=pltpu.CompilerParams(use_tc_tiling_on_sc=False),
    )


def _sc_agg(h, src3, dst3, zrows):
    hsplit = jnp.stack([h[:, :FH], h[:, FH:]])
    return _build_sc_agg()(hsplit, src3, dst3, zrows)


def _seg_sum(oh, v):
    return lax.dot_general(oh, v, (((0,), (0,)), ((), ())),
                           preferred_element_type=jnp.float32)


def _gather_g(oh, s):
    return jnp.dot(oh, s, preferred_element_type=jnp.float32)


def _norm(h, oh, w, b, a):
    cnt = jnp.maximum(jnp.sum(oh, axis=0), 1.0)[:, None]
    mean = _seg_sum(oh, h) / cnt
    sub = h - a * _gather_g(oh, mean)
    var = _seg_sum(oh, sub * sub) / cnt
    scale = w / jnp.sqrt(var + 1e-5)
    return sub * _gather_g(oh, scale) + b


def _matTt(x, wmat):
    return lax.dot_general(x, wmat, (((1,), (1,)), ((), ())),
                           preferred_element_type=jnp.float32)


def _k0_body(x_ref, oh_ref, w_ref, b_ref, a_ref, gn_ref):
    gn_ref[...] = _norm(x_ref[...], oh_ref[...],
                        w_ref[...], b_ref[...], a_ref[...])


def _mid_body(gn_ref, agg_ref, wr_ref, br_ref, wt_ref, oh_ref,
              w_ref, b_ref, a_ref, out_ref):
    agg2 = agg_ref[...]
    agg = jnp.concatenate([agg2[0, :N], agg2[1, :N]], axis=1)
    h = jnp.maximum(
        _matTt(agg, wr_ref[...]) + _matTt(gn_ref[...], wt_ref[...])
        + br_ref[...], 0.0)
    out_ref[...] = _norm(h, oh_ref[...], w_ref[...], b_ref[...], a_ref[...])


def _k3_body(gn_ref, agg_ref, wr_ref, br_ref, wt_ref, oh_ref,
             dw_ref, db_ref, ow_ref, ob_ref, out_ref):
    agg2 = agg_ref[...]
    agg = jnp.concatenate([agg2[0, :N], agg2[1, :N]], axis=1)
    h = jnp.maximum(
        _matTt(agg, wr_ref[...]) + _matTt(gn_ref[...], wt_ref[...])
        + br_ref[...], 0.0)
    oh = oh_ref[...]
    cnt = jnp.maximum(jnp.sum(oh, axis=0), 1.0)[:, None]
    pooled = _seg_sum(oh, h) / cnt
    d = jnp.maximum(_matTt(pooled, dw_ref[...]) + db_ref[...], 0.0)
    logits = _matTt(d, ow_ref[...]) + ob_ref[...]
    m = jnp.max(logits, axis=1, keepdims=True)
    e = jnp.exp(logits - m)
    out_ref[...] = e / jnp.sum(e, axis=1, keepdims=True)


def _tc(body, out_shape, *args):
    return pl.pallas_call(
        body, out_shape=jax.ShapeDtypeStruct(out_shape, jnp.float32))(*args)


def kernel(x, edge_index, batch, gn0_w, gn0_b, gn0_a, W1r, b1r, W1t,
           gn1_w, gn1_b, gn1_a, W2r, b2r, W2t, gn2_w, gn2_b, gn2_a,
           W3r, b3r, W3t, dW, db, oW, ob):
    oh = (batch[:, None] == jnp.arange(G, dtype=batch.dtype)[None, :]
          ).astype(jnp.float32)
    src3 = edge_index[0].reshape(NS, NCHUNK, CH)
    dst3 = edge_index[1].reshape(NS, NCHUNK, CH)
    zrows = jnp.zeros((RPT, FH), jnp.float32)
    r = lambda v: v.reshape(1, -1)

    gn1 = _tc(_k0_body, (N, F), x, oh, r(gn0_w), r(gn0_b), r(gn0_a))
    agg1 = _sc_agg(gn1, src3, dst3, zrows)
    gn2 = _tc(_mid_body, (N, F), gn1, agg1, W1r, r(b1r), W1t, oh,
              r(gn1_w), r(gn1_b), r(gn1_a))
    agg2 = _sc_agg(gn2, src3, dst3, zrows)
    gn3 = _tc(_mid_body, (N, F), gn2, agg2, W2r, r(b2r), W2t, oh,
              r(gn2_w), r(gn2_b), r(gn2_a))
    agg3 = _sc_agg(gn3, src3, dst3, zrows)
    return _tc(_k3_body, (G, C), gn3, agg3, W3r, r(b3r), W3t, oh,
               dW, r(db), oW, r(ob))


# speedup vs baseline: 11.2669x; 1.6607x over previous
import functools

import jax
import jax.numpy as jnp
from jax import lax
from jax.experimental import pallas as pl
from jax.experimental.pallas import tpu as pltpu
from jax.experimental.pallas import tpu_sc as plsc

N = 10000
E = 320000
F = 128
G = 8
C = 10

NC = 2
NS = 16
FH = F // NC
EPT = E // NS
CH = 100
NCHUNK = EPT // CH
NP = 10240
RPT = NP // NS

NBUF = 4


def _sc_agg_body(h_hbm, src_hbm, dst_hbm, zero_hbm, out_hbm,
                 src_v, dst_v, buf0, buf1, buf2, buf3, acc, gsem, ssem):
    cid = lax.axis_index("c")
    sid = lax.axis_index("s")
    bufs = (buf0, buf1, buf2, buf3)

    pltpu.sync_copy(zero_hbm, acc.at[pl.ds(sid * RPT, RPT)])
    pltpu.sync_copy(src_hbm.at[sid], src_v)
    pltpu.sync_copy(dst_hbm.at[sid], dst_v)
    plsc.subcore_barrier()

    hsrc = h_hbm.at[cid]

    def gather(i, slot):
        pltpu.async_copy(hsrc.at[src_v.at[i]], bufs[slot], gsem)

    def gather_wait(i, slot):
        pltpu.make_async_copy(hsrc.at[src_v.at[i]], bufs[slot], gsem).wait()

    def scatter(i, slot):
        pltpu.async_copy(bufs[slot], acc.at[dst_v.at[i]], ssem, add=True)

    def scatter_wait(i, slot):
        pltpu.make_async_copy(bufs[slot], acc.at[dst_v.at[i]], ssem).wait()

    for b in range(NBUF - 1):
        gather(b, b)

    def body(g, carry):
        for b in range(NBUF):
            i = g * NBUF + b
            sprev = (b - 1) % NBUF

            @pl.when(i > 0)
            def _():
                scatter_wait(i - 1, sprev)

            @pl.when(i + NBUF - 1 < NCHUNK)
            def _():
                gather(i + NBUF - 1, sprev)

            gather_wait(i, b)
            scatter(i, b)
        return carry

    lax.fori_loop(0, NCHUNK // NBUF, body, 0)
    scatter_wait(NCHUNK - 1, (NCHUNK - 1) % NBUF)

    plsc.subcore_barrier()
    pltpu.sync_copy(acc.at[pl.ds(sid * RPT, RPT)],
                    out_hbm.at[cid, pl.ds(sid * RPT, RPT)])


@functools.lru_cache(maxsize=1)
def _build_sc_agg():
    mesh = plsc.VectorSubcoreMesh(
        core_axis_name="c", subcore_axis_name="s",
        num_cores=NC, num_subcores=NS)
    return pl.kernel(
        _sc_agg_body,
        out_type=jax.ShapeDtypeStruct((NC, NP, FH), jnp.float32),
        mesh=mesh,
        scratch_types=[
            pltpu.VMEM((NCHUNK, CH), jnp.int32),
            pltpu.VMEM((NCHUNK, CH), jnp.int32),
            pltpu.VMEM((CH, FH), jnp.float32),
            pltpu.VMEM((CH, FH), jnp.float32),
            pltpu.VMEM((CH, FH), jnp.float32),
            pltpu.VMEM((CH, FH), jnp.float32),
            pltpu.VMEM_SHARED((NP, FH), jnp.float32),
            pltpu.SemaphoreType.DMA,
            pltpu.SemaphoreType.DMA,
        ],
        compiler_params=pltpu.CompilerParams(use_tc_tiling_on_sc=False),
    )


def _sc_agg(h, src3, dst3, zrows):
    hsplit = jnp.stack([h[:, :FH], h[:, FH:]])
    return _build_sc_agg()(hsplit, src3, dst3, zrows)


def _seg_sum(oh, v):
    return lax.dot_general(oh, v, (((0,), (0,)), ((), ())),
                           preferred_element_type=jnp.float32)


def _gather_g(oh, s):
    return jnp.dot(oh, s, preferred_element_type=jnp.float32)


def _norm(h, oh, w, b, a):
    cnt = jnp.maximum(jnp.sum(oh, axis=0), 1.0)[:, None]
    mean = _seg_sum(oh, h) / cnt
    sub = h - a * _gather_g(oh, mean)
    var = _seg_sum(oh, sub * sub) / cnt
    scale = w / jnp.sqrt(var + 1e-5)
    return sub * _gather_g(oh, scale) + b


def _matTt(x, wmat):
    return lax.dot_general(x, wmat, (((1,), (1,)), ((), ())),
                           preferred_element_type=jnp.float32)


def _k0_body(x_ref, oh_ref, w_ref, b_ref, a_ref, gn_ref):
    gn_ref[...] = _norm(x_ref[...], oh_ref[...],
                        w_ref[...], b_ref[...], a_ref[...])


def _mid_body(gn_ref, agg_ref, wr_ref, br_ref, wt_ref, oh_ref,
              w_ref, b_ref, a_ref, out_ref):
    agg2 = agg_ref[...]
    agg = jnp.concatenate([agg2[0, :N], agg2[1, :N]], axis=1)
    h = jnp.maximum(
        _matTt(agg, wr_ref[...]) + _matTt(gn_ref[...], wt_ref[...])
        + br_ref[...], 0.0)
    out_ref[...] = _norm(h, oh_ref[...], w_ref[...], b_ref[...], a_ref[...])


def _k3_body(gn_ref, agg_ref, wr_ref, br_ref, wt_ref, oh_ref,
             dw_ref, db_ref, ow_ref, ob_ref, out_ref):
    agg2 = agg_ref[...]
    agg = jnp.concatenate([agg2[0, :N], agg2[1, :N]], axis=1)
    h = jnp.maximum(
        _matTt(agg, wr_ref[...]) + _matTt(gn_ref[...], wt_ref[...])
        + br_ref[...], 0.0)
    oh = oh_ref[...]
    cnt = jnp.maximum(jnp.sum(oh, axis=0), 1.0)[:, None]
    pooled = _seg_sum(oh, h) / cnt
    d = jnp.maximum(_matTt(pooled, dw_ref[...]) + db_ref[...], 0.0)
    logits = _matTt(d, ow_ref[...]) + ob_ref[...]
    m = jnp.max(logits, axis=1, keepdims=True)
    e = jnp.exp(logits - m)
    out_ref[...] = e / jnp.sum(e, axis=1, keepdims=True)


def _tc(body, out_shape, *args):
    return pl.pallas_call(
        body, out_shape=jax.ShapeDtypeStruct(out_shape, jnp.float32))(*args)


def kernel(x, edge_index, batch, gn0_w, gn0_b, gn0_a, W1r, b1r, W1t,
           gn1_w, gn1_b, gn1_a, W2r, b2r, W2t, gn2_w, gn2_b, gn2_a,
           W3r, b3r, W3t, dW, db, oW, ob):
    oh = (batch[:, None] == jnp.arange(G, dtype=batch.dtype)[None, :]
          ).astype(jnp.float32)
    src3 = edge_index[0].reshape(NS, NCHUNK, CH)
    dst3 = edge_index[1].reshape(NS, NCHUNK, CH)
    zrows = jnp.zeros((RPT, FH), jnp.float32)
    r = lambda v: v.reshape(1, -1)

    gn1 = _tc(_k0_body, (N, F), x, oh, r(gn0_w), r(gn0_b), r(gn0_a))
    agg1 = _sc_agg(gn1, src3, dst3, zrows)
    gn2 = _tc(_mid_body, (N, F), gn1, agg1, W1r, r(b1r), W1t, oh,
              r(gn1_w), r(gn1_b), r(gn1_a))
    agg2 = _sc_agg(gn2, src3, dst3, zrows)
    gn3 = _tc(_mid_body, (N, F), gn2, agg2, W2r, r(b2r), W2t, oh,
              r(gn2_w), r(gn2_b), r(gn2_a))
    agg3 = _sc_agg(gn3, src3, dst3, zrows)
    return _tc(_k3_body, (G, C), gn3, agg3, W3r, r(b3r), W3t, oh,
               dW, r(db), oW, r(ob))


# speedup vs baseline: 11.3749x; 1.0096x over previous
import functools

import jax
import jax.numpy as jnp
from jax import lax
from jax.experimental import pallas as pl
from jax.experimental.pallas import tpu as pltpu
from jax.experimental.pallas import tpu_sc as plsc

N = 10000
E = 320000
F = 128
G = 8
C = 10

NC = 2
NS = 16
FH = F // NC
EPT = E // NS
CH = 125
NCHUNK = EPT // CH
NP = 10240
RPT = NP // NS

NBUF = 4
GA = 2
SW = 2


def _sc_agg_body(h_hbm, src_hbm, dst_hbm, zero_hbm, out_hbm,
                 src_v, dst_v, buf0, buf1, buf2, buf3, acc, gsem, ssem):
    cid = lax.axis_index("c")
    sid = lax.axis_index("s")
    bufs = (buf0, buf1, buf2, buf3)

    pltpu.sync_copy(zero_hbm, acc.at[pl.ds(sid * RPT, RPT)])
    pltpu.sync_copy(src_hbm.at[sid], src_v)
    pltpu.sync_copy(dst_hbm.at[sid], dst_v)
    plsc.subcore_barrier()

    hsrc = h_hbm.at[cid]

    def gather(i, slot):
        pltpu.async_copy(hsrc.at[src_v.at[i]], bufs[slot], gsem)

    def gather_wait(i, slot):
        pltpu.make_async_copy(hsrc.at[src_v.at[i]], bufs[slot], gsem).wait()

    def scatter(i, slot):
        pltpu.async_copy(bufs[slot], acc.at[dst_v.at[i]], ssem, add=True)

    def scatter_wait(i, slot):
        pltpu.make_async_copy(bufs[slot], acc.at[dst_v.at[i]], ssem).wait()

    for b in range(GA):
        gather(b, b)

    def body(g, carry):
        for b in range(NBUF):
            i = g * NBUF + b

            @pl.when(i >= SW)
            def _():
                scatter_wait(i - SW, (b - SW) % NBUF)

            @pl.when(i + GA < NCHUNK)
            def _():
                gather(i + GA, (b + GA) % NBUF)

            gather_wait(i, b)
            scatter(i, b)
        return carry

    lax.fori_loop(0, NCHUNK // NBUF, body, 0)
    for k in range(SW):
        scatter_wait(NCHUNK - SW + k, (NCHUNK - SW + k) % NBUF)

    plsc.subcore_barrier()
    pltpu.sync_copy(acc.at[pl.ds(sid * RPT, RPT)],
                    out_hbm.at[cid, pl.ds(sid * RPT, RPT)])


@functools.lru_cache(maxsize=1)
def _build_sc_agg():
    mesh = plsc.VectorSubcoreMesh(
        core_axis_name="c", subcore_axis_name="s",
        num_cores=NC, num_subcores=NS)
    return pl.kernel(
        _sc_agg_body,
        out_type=jax.ShapeDtypeStruct((NC, NP, FH), jnp.float32),
        mesh=mesh,
        scratch_types=[
            pltpu.VMEM((NCHUNK, CH), jnp.int32),
            pltpu.VMEM((NCHUNK, CH), jnp.int32),
            *([pltpu.VMEM((CH, FH), jnp.float32)] * NBUF),
            pltpu.VMEM_SHARED((NP, FH), jnp.float32),
            pltpu.SemaphoreType.DMA,
            pltpu.SemaphoreType.DMA,
        ],
        compiler_params=pltpu.CompilerParams(use_tc_tiling_on_sc=False),
    )


def _sc_agg(h, src3, dst3, zrows):
    hsplit = jnp.stack([h[:, :FH], h[:, FH:]])
    return _build_sc_agg()(hsplit, src3, dst3, zrows)


def _seg_sum(oh, v):
    return lax.dot_general(oh, v, (((0,), (0,)), ((), ())),
                           preferred_element_type=jnp.float32)


def _gather_g(oh, s):
    return jnp.dot(oh, s, preferred_element_type=jnp.float32)


def _norm(h, oh, w, b, a):
    cnt = jnp.maximum(jnp.sum(oh, axis=0), 1.0)[:, None]
    mean = _seg_sum(oh, h) / cnt
    sub = h - a * _gather_g(oh, mean)
    var = _seg_sum(oh, sub * sub) / cnt
    scale = w / jnp.sqrt(var + 1e-5)
    return sub * _gather_g(oh, scale) + b


def _matTt(x, wmat):
    return lax.dot_general(x, wmat, (((1,), (1,)), ((), ())),
                           preferred_element_type=jnp.float32)


def _k0_body(x_ref, oh_ref, w_ref, b_ref, a_ref, gn_ref):
    gn_ref[...] = _norm(x_ref[...], oh_ref[...],
                        w_ref[...], b_ref[...], a_ref[...])


def _mid_body(gn_ref, agg_ref, wr_ref, br_ref, wt_ref, oh_ref,
              w_ref, b_ref, a_ref, out_ref):
    agg2 = agg_ref[...]
    agg = jnp.concatenate([agg2[0, :N], agg2[1, :N]], axis=1)
    h = jnp.maximum(
        _matTt(agg, wr_ref[...]) + _matTt(gn_ref[...], wt_ref[...])
        + br_ref[...], 0.0)
    out_ref[...] = _norm(h, oh_ref[...], w_ref[...], b_ref[...], a_ref[...])


def _k3_body(gn_ref, agg_ref, wr_ref, br_ref, wt_ref, oh_ref,
             dw_ref, db_ref, ow_ref, ob_ref, out_ref):
    agg2 = agg_ref[...]
    agg = jnp.concatenate([agg2[0, :N], agg2[1, :N]], axis=1)
    h = jnp.maximum(
        _matTt(agg, wr_ref[...]) + _matTt(gn_ref[...], wt_ref[...])
        + br_ref[...], 0.0)
    oh = oh_ref[...]
    cnt = jnp.maximum(jnp.sum(oh, axis=0), 1.0)[:, None]
    pooled = _seg_sum(oh, h) / cnt
    d = jnp.maximum(_matTt(pooled, dw_ref[...]) + db_ref[...], 0.0)
    logits = _matTt(d, ow_ref[...]) + ob_ref[...]
    m = jnp.max(logits, axis=1, keepdims=True)
    e = jnp.exp(logits - m)
    out_ref[...] = e / jnp.sum(e, axis=1, keepdims=True)


def _tc(body, out_shape, *args):
    return pl.pallas_call(
        body, out_shape=jax.ShapeDtypeStruct(out_shape, jnp.float32))(*args)


def kernel(x, edge_index, batch, gn0_w, gn0_b, gn0_a, W1r, b1r, W1t,
           gn1_w, gn1_b, gn1_a, W2r, b2r, W2t, gn2_w, gn2_b, gn2_a,
           W3r, b3r, W3t, dW, db, oW, ob):
    oh = (batch[:, None] == jnp.arange(G, dtype=batch.dtype)[None, :]
          ).astype(jnp.float32)
    src3 = edge_index[0].reshape(NS, NCHUNK, CH)
    dst3 = edge_index[1].reshape(NS, NCHUNK, CH)
    zrows = jnp.zeros((RPT, FH), jnp.float32)
    r = lambda v: v.reshape(1, -1)

    gn1 = _tc(_k0_body, (N, F), x, oh, r(gn0_w), r(gn0_b), r(gn0_a))
    agg1 = _sc_agg(gn1, src3, dst3, zrows)
    gn2 = _tc(_mid_body, (N, F), gn1, agg1, W1r, r(b1r), W1t, oh,
              r(gn1_w), r(gn1_b), r(gn1_a))
    agg2 = _sc_agg(gn2, src3, dst3, zrows)
    gn3 = _tc(_mid_body, (N, F), gn2, agg2, W2r, r(b2r), W2t, oh,
              r(gn2_w), r(gn2_b), r(gn2_a))
    agg3 = _sc_agg(gn3, src3, dst3, zrows)
    return _tc(_k3_body, (G, C), gn3, agg3, W3r, r(b3r), W3t, oh,
               dW, r(db), oW, r(ob))


# speedup vs baseline: 12.2222x; 1.0745x over previous
import functools

import jax
import jax.numpy as jnp
from jax import lax
from jax.experimental import pallas as pl
from jax.experimental.pallas import tpu as pltpu
from jax.experimental.pallas import tpu_sc as plsc

N = 10000
E = 320000
F = 128
G = 8
C = 10

NC = 2
NS = 16
FH = F // NC
EPT = E // NS
CH = 125
NCHUNK = EPT // CH
NP = 10240
RPT = NP // NS

NBUF = 4
GA = 2
SW = 2


def _sc_agg_body(h_hbm, src_hbm, dst_hbm, zero_hbm, out_hbm,
                 src_v, dst_v, buf0, buf1, buf2, buf3, acc, gsem, ssem):
    cid = lax.axis_index("c")
    sid = lax.axis_index("s")
    bufs = (buf0, buf1, buf2, buf3)

    pltpu.sync_copy(zero_hbm, acc.at[pl.ds(sid * RPT, RPT)])
    pltpu.sync_copy(src_hbm.at[sid], src_v)
    pltpu.sync_copy(dst_hbm.at[sid], dst_v)
    plsc.subcore_barrier()

    hsrc = h_hbm.at[cid]

    def gather(i, slot):
        pltpu.async_copy(hsrc.at[src_v.at[i]], bufs[slot], gsem)

    def gather_wait(i, slot):
        pltpu.make_async_copy(hsrc.at[src_v.at[i]], bufs[slot], gsem).wait()

    def scatter(i, slot):
        pltpu.async_copy(bufs[slot], acc.at[dst_v.at[i]], ssem, add=True)

    def scatter_wait(i, slot):
        pltpu.make_async_copy(bufs[slot], acc.at[dst_v.at[i]], ssem).wait()

    for b in range(GA):
        gather(b, b)

    def body(g, carry):
        for b in range(NBUF):
            i = g * NBUF + b

            @pl.when(i >= SW)
            def _():
                scatter_wait(i - SW, (b - SW) % NBUF)

            @pl.when(i + GA < NCHUNK)
            def _():
                gather(i + GA, (b + GA) % NBUF)

            gather_wait(i, b)
            scatter(i, b)
        return carry

    lax.fori_loop(0, NCHUNK // NBUF, body, 0)
    for k in range(SW):
        scatter_wait(NCHUNK - SW + k, (NCHUNK - SW + k) % NBUF)

    plsc.subcore_barrier()
    pltpu.sync_copy(acc.at[pl.ds(sid * RPT, RPT)],
                    out_hbm.at[cid, pl.ds(sid * RPT, RPT)])


@functools.lru_cache(maxsize=1)
def _build_sc_agg():
    mesh = plsc.VectorSubcoreMesh(
        core_axis_name="c", subcore_axis_name="s",
        num_cores=NC, num_subcores=NS)
    return pl.kernel(
        _sc_agg_body,
        out_type=jax.ShapeDtypeStruct((NC, NP, FH), jnp.float32),
        mesh=mesh,
        scratch_types=[
            pltpu.VMEM((NCHUNK, CH), jnp.int32),
            pltpu.VMEM((NCHUNK, CH), jnp.int32),
            *([pltpu.VMEM((CH, FH), jnp.float32)] * NBUF),
            pltpu.VMEM_SHARED((NP, FH), jnp.float32),
            pltpu.SemaphoreType.DMA,
            pltpu.SemaphoreType.DMA,
        ],
        compiler_params=pltpu.CompilerParams(use_tc_tiling_on_sc=False),
    )


def _sc_agg(hsplit, src3, dst3, zrows):
    return _build_sc_agg()(hsplit, src3, dst3, zrows)


def _seg_sum(oht, v):
    return jnp.dot(oht, v, preferred_element_type=jnp.float32)


def _gather_g(oht, s):
    return lax.dot_general(oht, s, (((0,), (0,)), ((), ())),
                           preferred_element_type=jnp.float32)


def _split(h):
    return jnp.stack([h[:, :FH], h[:, FH:]])


def _norm(h, oht, w, b, a):
    cnt = jnp.maximum(jnp.sum(oht, axis=1), 1.0)[:, None]
    mean = _seg_sum(oht, h) / cnt
    sub = h - a * _gather_g(oht, mean)
    var = _seg_sum(oht, sub * sub) / cnt
    scale = w / jnp.sqrt(var + 1e-5)
    return sub * _gather_g(oht, scale) + b


def _matTt(x, wmat):
    return lax.dot_general(x, wmat, (((1,), (1,)), ((), ())),
                           preferred_element_type=jnp.float32)


def _k0_body(x_ref, oh_ref, w_ref, b_ref, a_ref, gn_ref):
    gn_ref[...] = _split(_norm(x_ref[...], oh_ref[...],
                               w_ref[...], b_ref[...], a_ref[...]))


def _mid_body(gn_ref, agg_ref, wr_ref, br_ref, wt_ref, oh_ref,
              w_ref, b_ref, a_ref, out_ref):
    agg2 = agg_ref[...]
    agg = jnp.concatenate([agg2[0, :N], agg2[1, :N]], axis=1)
    gs = gn_ref[...]
    gn = jnp.concatenate([gs[0], gs[1]], axis=1)
    h = jnp.maximum(
        _matTt(agg, wr_ref[...]) + _matTt(gn, wt_ref[...])
        + br_ref[...], 0.0)
    out_ref[...] = _split(_norm(h, oh_ref[...], w_ref[...], b_ref[...],
                                a_ref[...]))


def _k3_body(gn_ref, agg_ref, wr_ref, br_ref, wt_ref, oh_ref,
             dw_ref, db_ref, ow_ref, ob_ref, out_ref):
    agg2 = agg_ref[...]
    agg = jnp.concatenate([agg2[0, :N], agg2[1, :N]], axis=1)
    gs = gn_ref[...]
    gn = jnp.concatenate([gs[0], gs[1]], axis=1)
    h = jnp.maximum(
        _matTt(agg, wr_ref[...]) + _matTt(gn, wt_ref[...])
        + br_ref[...], 0.0)
    oh = oh_ref[...]
    cnt = jnp.maximum(jnp.sum(oh, axis=1), 1.0)[:, None]
    pooled = _seg_sum(oh, h) / cnt
    d = jnp.maximum(_matTt(pooled, dw_ref[...]) + db_ref[...], 0.0)
    logits = _matTt(d, ow_ref[...]) + ob_ref[...]
    m = jnp.max(logits, axis=1, keepdims=True)
    e = jnp.exp(logits - m)
    out_ref[...] = e / jnp.sum(e, axis=1, keepdims=True)


def _tc(body, out_shape, *args):
    return pl.pallas_call(
        body, out_shape=jax.ShapeDtypeStruct(out_shape, jnp.float32))(*args)


def kernel(x, edge_index, batch, gn0_w, gn0_b, gn0_a, W1r, b1r, W1t,
           gn1_w, gn1_b, gn1_a, W2r, b2r, W2t, gn2_w, gn2_b, gn2_a,
           W3r, b3r, W3t, dW, db, oW, ob):
    oh = (batch[None, :] == jnp.arange(G, dtype=batch.dtype)[:, None]
          ).astype(jnp.float32)
    src3 = edge_index[0].reshape(NS, NCHUNK, CH)
    dst3 = edge_index[1].reshape(NS, NCHUNK, CH)
    zrows = jnp.zeros((RPT, FH), jnp.float32)
    r = lambda v: v.reshape(1, -1)

    gn1 = _tc(_k0_body, (NC, N, FH), x, oh, r(gn0_w), r(gn0_b), r(gn0_a))
    agg1 = _sc_agg(gn1, src3, dst3, zrows)
    gn2 = _tc(_mid_body, (NC, N, FH), gn1, agg1, W1r, r(b1r), W1t, oh,
              r(gn1_w), r(gn1_b), r(gn1_a))
    agg2 = _sc_agg(gn2, src3, dst3, zrows)
    gn3 = _tc(_mid_body, (NC, N, FH), gn2, agg2, W2r, r(b2r), W2t, oh,
              r(gn2_w), r(gn2_b), r(gn2_a))
    agg3 = _sc_agg(gn3, src3, dst3, zrows)
    return _tc(_k3_body, (G, C), gn3, agg3, W3r, r(b3r), W3t, oh,
               dW, r(db), oW, r(ob))
